# speedup vs baseline: 1.8430x; 1.8430x over previous
import functools

import jax
import jax.numpy as jnp
from jax import lax
from jax.experimental import pallas as pl
from jax.experimental.pallas import tpu as pltpu
from jax.experimental.pallas import tpu_sc as plsc

N_SPEAKERS = 100000
SIZE = 8192
BATCH = 4096

_info = plsc.get_sparse_core_info()
_NC = _info.num_cores
_NS = _info.num_subcores
_NW = _NC * _NS
_BPW = BATCH // _NW
_K = 4
_NSTEPS = _BPW // _K


def _gather_kernel(idx_hbm, table_hbm, out_hbm, idx_v, buf0, buf1,
                   gs0, gs1, ws0, ws1):
    wid = lax.axis_index("s") * _NC + lax.axis_index("c")
    base = wid * _BPW

    pltpu.sync_copy(idx_hbm.at[wid], idx_v)

    bufs = (buf0, buf1)
    gsems = (gs0, gs1)
    wsems = (ws0, ws1)
    gathers = [None, None]
    writes = [None, None]

    gathers[0] = pltpu.async_copy(table_hbm.at[idx_v.at[0]], bufs[0], gsems[0])
    for j in range(_NSTEPS):
        b = j % 2
        nb = (j + 1) % 2
        if j + 1 < _NSTEPS:
            if writes[nb] is not None:
                writes[nb].wait()
            gathers[nb] = pltpu.async_copy(
                table_hbm.at[idx_v.at[j + 1]], bufs[nb], gsems[nb])
        gathers[b].wait()
        writes[b] = pltpu.async_copy(
            bufs[b], out_hbm.at[pl.ds(base + j * _K, _K)], wsems[b])
    writes[0].wait()
    writes[1].wait()


@jax.jit
def _run(sid32, emb):
    mesh = plsc.VectorSubcoreMesh(core_axis_name="c", subcore_axis_name="s")
    return pl.kernel(
        _gather_kernel,
        mesh=mesh,
        out_type=jax.ShapeDtypeStruct((BATCH, SIZE), jnp.float32),
        scratch_types=[
            pltpu.VMEM((_NSTEPS, _K), jnp.int32),
            pltpu.VMEM((_K, SIZE), jnp.float32),
            pltpu.VMEM((_K, SIZE), jnp.float32),
            pltpu.SemaphoreType.DMA,
            pltpu.SemaphoreType.DMA,
            pltpu.SemaphoreType.DMA,
            pltpu.SemaphoreType.DMA,
        ],
    )(sid32, emb)


def kernel(sid, emb):
    sid32 = sid.astype(jnp.int32).reshape(_NW, _NSTEPS, _K)
    return _run(sid32, emb)


# speedup vs baseline: 1.8453x; 1.0012x over previous
import functools

import jax
import jax.numpy as jnp
from jax import lax
from jax.experimental import pallas as pl
from jax.experimental.pallas import tpu as pltpu
from jax.experimental.pallas import tpu_sc as plsc

N_SPEAKERS = 100000
SIZE = 8192
BATCH = 4096

_info = plsc.get_sparse_core_info()
_NC = _info.num_cores
_NS = _info.num_subcores
_NW = _NC * _NS
_BPW = BATCH // _NW
_K = 4
_NBUF = 3
_NSTEPS = _BPW // _K


def _gather_kernel(idx_hbm, table_hbm, out_hbm, idx_v, *bufs_and_sems):
    bufs = bufs_and_sems[:_NBUF]
    gsems = bufs_and_sems[_NBUF:2 * _NBUF]
    wsems = bufs_and_sems[2 * _NBUF:3 * _NBUF]

    wid = lax.axis_index("s") * _NC + lax.axis_index("c")
    base = wid * _BPW

    pltpu.sync_copy(idx_hbm.at[wid], idx_v)

    gathers = [None] * _NBUF
    writes = [None] * _NBUF

    for j in range(_NBUF - 1):
        gathers[j] = pltpu.async_copy(
            table_hbm.at[idx_v.at[j]], bufs[j], gsems[j])

    for j in range(_NSTEPS):
        b = j % _NBUF
        pb = (j - 1) % _NBUF
        if writes[pb] is not None:
            writes[pb].wait()
        if j + _NBUF - 1 < _NSTEPS:
            gathers[pb] = pltpu.async_copy(
                table_hbm.at[idx_v.at[j + _NBUF - 1]], bufs[pb], gsems[pb])
        gathers[b].wait()
        writes[b] = pltpu.async_copy(
            bufs[b], out_hbm.at[pl.ds(base + j * _K, _K)], wsems[b])

    writes[(_NSTEPS - 1) % _NBUF].wait()


@jax.jit
def _run(sid32, emb):
    mesh = plsc.VectorSubcoreMesh(core_axis_name="c", subcore_axis_name="s")
    return pl.kernel(
        _gather_kernel,
        mesh=mesh,
        out_type=jax.ShapeDtypeStruct((BATCH, SIZE), jnp.float32),
        scratch_types=(
            [pltpu.VMEM((_NSTEPS, _K), jnp.int32)]
            + [pltpu.VMEM((_K, SIZE), jnp.float32) for _ in range(_NBUF)]
            + [pltpu.SemaphoreType.DMA for _ in range(2 * _NBUF)]
        ),
    )(sid32, emb)


def kernel(sid, emb):
    sid32 = sid.astype(jnp.int32).reshape(_NW, _NSTEPS, _K)
    return _run(sid32, emb)
